# baseline (speedup 1.0000x reference)
import functools

import jax
import jax.numpy as jnp
from jax import lax
from jax.experimental import pallas as pl
from jax.experimental.pallas import tpu as pltpu
from jax.experimental.pallas import tpu_sc as plsc

N_NODES = 10000
EMBED = 128
BT = 24

NC = 2
NS = 16
NW = NC * NS
N_PAD = 10240
ROWS_PER_W = N_PAD // NW
CHUNK = 80
NCHUNKS = ROWS_PER_W // CHUNK


def _sc_gather_body(zin_hbm, zout_hbm, din_hbm, dout_hbm, out_hbm,
                    idx_in_v, idx_out_v, rows_in_v, rows_out_v, sem):
    wid = lax.axis_index("s") * NC + lax.axis_index("c")
    base = wid * ROWS_PER_W
    cps = []
    for j in range(NCHUNKS):
        off = base + j * CHUNK
        cps.append(pltpu.async_copy(din_hbm.at[pl.ds(off, CHUNK)],
                                    idx_in_v.at[j], sem))
        cps.append(pltpu.async_copy(dout_hbm.at[pl.ds(off, CHUNK)],
                                    idx_out_v.at[j], sem))
    for cp in cps:
        cp.wait()
    cps = []
    for j in range(NCHUNKS):
        sl = pl.ds(j * CHUNK, CHUNK)
        cps.append(pltpu.async_copy(zin_hbm.at[idx_in_v.at[j]],
                                    rows_in_v.at[sl], sem))
        cps.append(pltpu.async_copy(zout_hbm.at[idx_out_v.at[j]],
                                    rows_out_v.at[sl], sem))
    for cp in cps:
        cp.wait()
    cps = [pltpu.async_copy(rows_in_v, out_hbm.at[0, pl.ds(base, ROWS_PER_W)], sem),
           pltpu.async_copy(rows_out_v, out_hbm.at[1, pl.ds(base, ROWS_PER_W)], sem)]
    for cp in cps:
        cp.wait()


EMBED_PK = EMBED // 2

_sc_gather = functools.partial(
    pl.kernel,
    out_type=jax.ShapeDtypeStruct((2, N_PAD, EMBED_PK), jnp.int32),
    mesh=plsc.VectorSubcoreMesh(core_axis_name="c", subcore_axis_name="s"),
    compiler_params=pltpu.CompilerParams(use_tc_tiling_on_sc=False),
    scratch_types=[
        pltpu.VMEM((NCHUNKS, CHUNK), jnp.int32),
        pltpu.VMEM((NCHUNKS, CHUNK), jnp.int32),
        pltpu.VMEM((ROWS_PER_W, EMBED_PK), jnp.int32),
        pltpu.VMEM((ROWS_PER_W, EMBED_PK), jnp.int32),
        pltpu.SemaphoreType.DMA,
    ],
)(_sc_gather_body)


def _add_body(x_ref, c_ref, o_ref):
    cent = c_ref[0].astype(jnp.float32) + c_ref[1].astype(jnp.float32)
    o_ref[...] = x_ref[...] + cent[None]


def _tc_add(xr, cent2, block_n, block_bt=BT):
    nb = N_NODES // block_n
    nbt = BT // block_bt
    return pl.pallas_call(
        _add_body,
        grid=(nb, nbt),
        in_specs=[
            pl.BlockSpec((block_bt, block_n, EMBED), lambda n, bt: (bt, n, 0)),
            pl.BlockSpec((2, block_n, EMBED), lambda n, bt: (0, n, 0)),
        ],
        out_specs=pl.BlockSpec((block_bt, block_n, EMBED), lambda n, bt: (bt, n, 0)),
        out_shape=jax.ShapeDtypeStruct((BT, N_NODES, EMBED), jnp.float32),
    )(xr, cent2)


def kernel(x, z_in, z_out, in_degree, out_degree):
    din = jnp.pad(in_degree.astype(jnp.int32), (0, N_PAD - N_NODES))
    dout = jnp.pad(out_degree.astype(jnp.int32), (0, N_PAD - N_NODES))

    def pack(z):
        zb = z.astype(jnp.bfloat16).reshape(z.shape[0], EMBED_PK, 2)
        return lax.bitcast_convert_type(zb, jnp.int32)

    cent_pk = _sc_gather(pack(z_in), pack(z_out), din, dout)
    cent2 = lax.bitcast_convert_type(cent_pk, jnp.bfloat16).reshape(
        2, N_PAD, EMBED)
    xr = x.reshape(BT, N_NODES, EMBED)
    out = _tc_add(xr, cent2, 2000, 12)
    return out.reshape(x.shape)


# speedup vs baseline: 1.3372x; 1.3372x over previous
import functools

import jax
import jax.numpy as jnp
from jax import lax
from jax.experimental import pallas as pl
from jax.experimental.pallas import tpu as pltpu
from jax.experimental.pallas import tpu_sc as plsc

N_NODES = 10000
EMBED = 128
BT = 24

NC = 2
NS = 16
NW = NC * NS
N_PAD = 10240
ROWS_PER_W = N_PAD // NW
CHUNKS = (128, 128, 64)
NCHUNKS = len(CHUNKS)
MAXCHUNK = max(CHUNKS)
OFFS = (0, 128, 256)


def _sc_gather_body(zin_hbm, zout_hbm, din_hbm, dout_hbm, out_hbm,
                    idx_in_v, idx_out_v, rows_in_v, rows_out_v, sem):
    wid = lax.axis_index("s") * NC + lax.axis_index("c")
    base = wid * ROWS_PER_W
    cps = []
    for j, (o, ch) in enumerate(zip(OFFS, CHUNKS)):
        cps.append(pltpu.async_copy(din_hbm.at[pl.ds(base + o, ch)],
                                    idx_in_v.at[j, pl.ds(0, ch)], sem))
        cps.append(pltpu.async_copy(dout_hbm.at[pl.ds(base + o, ch)],
                                    idx_out_v.at[j, pl.ds(0, ch)], sem))
    for cp in cps:
        cp.wait()
    cps = []
    for j, (o, ch) in enumerate(zip(OFFS, CHUNKS)):
        sl = pl.ds(o, ch)
        cps.append(pltpu.async_copy(zin_hbm.at[idx_in_v.at[j, pl.ds(0, ch)]],
                                    rows_in_v.at[sl], sem))
        cps.append(pltpu.async_copy(zout_hbm.at[idx_out_v.at[j, pl.ds(0, ch)]],
                                    rows_out_v.at[sl], sem))
    for cp in cps:
        cp.wait()
    cps = [pltpu.async_copy(rows_in_v, out_hbm.at[0, pl.ds(base, ROWS_PER_W)], sem),
           pltpu.async_copy(rows_out_v, out_hbm.at[1, pl.ds(base, ROWS_PER_W)], sem)]
    for cp in cps:
        cp.wait()


_sc_gather = functools.partial(
    pl.kernel,
    out_type=jax.ShapeDtypeStruct((2, N_PAD, EMBED), jnp.float32),
    mesh=plsc.VectorSubcoreMesh(core_axis_name="c", subcore_axis_name="s"),
    scratch_types=[
        pltpu.VMEM((NCHUNKS, MAXCHUNK), jnp.int32),
        pltpu.VMEM((NCHUNKS, MAXCHUNK), jnp.int32),
        pltpu.VMEM((ROWS_PER_W, EMBED), jnp.float32),
        pltpu.VMEM((ROWS_PER_W, EMBED), jnp.float32),
        pltpu.SemaphoreType.DMA,
    ],
)(_sc_gather_body)


def _add_body(x_ref, c_ref, o_ref):
    o_ref[...] = x_ref[...] + (c_ref[0] + c_ref[1])[None]


def _tc_add(xr, cent2, block_n, block_bt=BT):
    nb = N_NODES // block_n
    nbt = BT // block_bt
    return pl.pallas_call(
        _add_body,
        grid=(nb, nbt),
        in_specs=[
            pl.BlockSpec((block_bt, block_n, EMBED), lambda n, bt: (bt, n, 0)),
            pl.BlockSpec((2, block_n, EMBED), lambda n, bt: (0, n, 0)),
        ],
        out_specs=pl.BlockSpec((block_bt, block_n, EMBED), lambda n, bt: (bt, n, 0)),
        out_shape=jax.ShapeDtypeStruct((BT, N_NODES, EMBED), jnp.float32),
    )(xr, cent2)


def kernel(x, z_in, z_out, in_degree, out_degree):
    din = jnp.pad(in_degree.astype(jnp.int32), (0, N_PAD - N_NODES))
    dout = jnp.pad(out_degree.astype(jnp.int32), (0, N_PAD - N_NODES))
    cent2 = _sc_gather(z_in, z_out, din, dout)
    xr = x.reshape(BT, N_NODES, EMBED)
    out = _tc_add(xr, cent2, 2000, 12)
    return out.reshape(x.shape)
